# initial kernel scaffold (unmeasured)
import jax
import jax.numpy as jnp
from jax import lax
from jax.experimental import pallas as pl
from jax.experimental.pallas import tpu as pltpu

N_LOCAL_EXPERTS = 2


def kernel(x, assign, W1, W2):
    t, d = x.shape
    assign_col = assign.reshape(t, 1)

    def body(x_ref, a_ref, w1_ref, w2_ref, out_ref,
             xr_ref, ar_ref, psend_ref, precv_ref, send_sems, recv_sems):
        my_x = lax.axis_index("x")
        my_y = lax.axis_index("y")
        my_z = lax.axis_index("z")
        partner = (1 - my_x, my_y, my_z)

        barrier = pltpu.get_barrier_semaphore()
        pl.semaphore_signal(barrier, inc=1, device_id=partner,
                            device_id_type=pl.DeviceIdType.MESH)
        pl.semaphore_wait(barrier, 1)

        send_x = pltpu.make_async_remote_copy(
            src_ref=x_ref, dst_ref=xr_ref,
            send_sem=send_sems.at[0], recv_sem=recv_sems.at[0],
            device_id=partner, device_id_type=pl.DeviceIdType.MESH)
        send_x.start()
        send_a = pltpu.make_async_remote_copy(
            src_ref=a_ref, dst_ref=ar_ref,
            send_sem=send_sems.at[1], recv_sem=recv_sems.at[1],
            device_id=partner, device_id_type=pl.DeviceIdType.MESH)
        send_a.start()

        e0 = my_x * N_LOCAL_EXPERTS

        def local_experts(xs, acol):
            acc = jnp.zeros((t, d), jnp.float32)
            for j in range(N_LOCAL_EXPERTS):
                m = (acol == e0 + j).astype(jnp.float32)
                h = jnp.maximum(
                    jnp.dot(xs * m, w1_ref[j],
                            preferred_element_type=jnp.float32),
                    0.0)
                acc = acc + jnp.dot(h, w2_ref[j],
                                    preferred_element_type=jnp.float32)
            return acc

        out_ref[:, :] = local_experts(x_ref[:, :], a_ref[:, :])

        send_x.wait()
        send_a.wait()

        psend_ref[:, :] = local_experts(xr_ref[:, :], ar_ref[:, :])
        send_p = pltpu.make_async_remote_copy(
            src_ref=psend_ref, dst_ref=precv_ref,
            send_sem=send_sems.at[2], recv_sem=recv_sems.at[2],
            device_id=partner, device_id_type=pl.DeviceIdType.MESH)
        send_p.start()
        send_p.wait()

        out_ref[:, :] = out_ref[:, :] + precv_ref[:, :]

    return pl.pallas_call(
        body,
        out_shape=jax.ShapeDtypeStruct((t, d), jnp.float32),
        in_specs=[pl.BlockSpec(memory_space=pltpu.VMEM)] * 4,
        out_specs=pl.BlockSpec(memory_space=pltpu.VMEM),
        scratch_shapes=[
            pltpu.VMEM((t, d), jnp.float32),
            pltpu.VMEM((t, 1), jnp.int32),
            pltpu.VMEM((t, d), jnp.float32),
            pltpu.VMEM((t, d), jnp.float32),
            pltpu.SemaphoreType.DMA((3,)),
            pltpu.SemaphoreType.DMA((3,)),
        ],
        compiler_params=pltpu.CompilerParams(collective_id=0),
    )(x, assign_col, W1, W2)


# baseline (device time: 149551 ns/iter reference)
import jax
import jax.numpy as jnp
from jax import lax
from jax.experimental import pallas as pl
from jax.experimental.pallas import tpu as pltpu

N_LOCAL_EXPERTS = 2


def kernel(x, assign, W1, W2):
    t, d = x.shape
    assign_col = assign.reshape(t, 1)

    def body(x_ref, a_ref, w1_ref, w2_ref, out_ref,
             xr_ref, ar_ref, psend_ref, precv_ref, w1s_ref, w2s_ref,
             wsems, send_sems, recv_sems):
        my_x = lax.axis_index("x")
        my_y = lax.axis_index("y")
        my_z = lax.axis_index("z")
        partner = (1 - my_x, my_y, my_z)

        barrier = pltpu.get_barrier_semaphore()
        pl.semaphore_signal(barrier, inc=1, device_id=partner,
                            device_id_type=pl.DeviceIdType.MESH)
        pl.semaphore_wait(barrier, 1)

        send_x = pltpu.make_async_remote_copy(
            src_ref=x_ref, dst_ref=xr_ref,
            send_sem=send_sems.at[0], recv_sem=recv_sems.at[0],
            device_id=partner, device_id_type=pl.DeviceIdType.MESH)
        send_x.start()
        send_a = pltpu.make_async_remote_copy(
            src_ref=a_ref, dst_ref=ar_ref,
            send_sem=send_sems.at[1], recv_sem=recv_sems.at[1],
            device_id=partner, device_id_type=pl.DeviceIdType.MESH)
        send_a.start()

        e0 = my_x * N_LOCAL_EXPERTS

        def expert_partial(xs, acol, j):
            m = (acol == e0 + j).astype(jnp.float32)
            h = jnp.maximum(
                jnp.dot(xs * m, w1s_ref[:, :],
                        preferred_element_type=jnp.float32),
                0.0)
            return jnp.dot(h, w2s_ref[:, :],
                           preferred_element_type=jnp.float32)

        for j in range(N_LOCAL_EXPERTS):
            cp1 = pltpu.make_async_copy(w1_ref.at[j], w1s_ref, wsems.at[0])
            cp2 = pltpu.make_async_copy(w2_ref.at[j], w2s_ref, wsems.at[1])
            cp1.start()
            cp2.start()
            cp1.wait()
            cp2.wait()

            pj = expert_partial(x_ref[:, :], a_ref[:, :], j)
            if j == 0:
                out_ref[:, :] = pj
                send_x.wait()
                send_a.wait()
            else:
                out_ref[:, :] = out_ref[:, :] + pj

            pr = expert_partial(xr_ref[:, :], ar_ref[:, :], j)
            if j == 0:
                psend_ref[:, :] = pr
            else:
                psend_ref[:, :] = psend_ref[:, :] + pr

        send_p = pltpu.make_async_remote_copy(
            src_ref=psend_ref, dst_ref=precv_ref,
            send_sem=send_sems.at[2], recv_sem=recv_sems.at[2],
            device_id=partner, device_id_type=pl.DeviceIdType.MESH)
        send_p.start()
        send_p.wait()

        out_ref[:, :] = out_ref[:, :] + precv_ref[:, :]

    _, dk, f = W1.shape
    return pl.pallas_call(
        body,
        out_shape=jax.ShapeDtypeStruct((t, d), jnp.float32),
        in_specs=[
            pl.BlockSpec(memory_space=pltpu.VMEM),
            pl.BlockSpec(memory_space=pltpu.VMEM),
            pl.BlockSpec(memory_space=pltpu.MemorySpace.HBM),
            pl.BlockSpec(memory_space=pltpu.MemorySpace.HBM),
        ],
        out_specs=pl.BlockSpec(memory_space=pltpu.VMEM),
        scratch_shapes=[
            pltpu.VMEM((t, d), jnp.float32),
            pltpu.VMEM((t, 1), jnp.int32),
            pltpu.VMEM((t, d), jnp.float32),
            pltpu.VMEM((t, d), jnp.float32),
            pltpu.VMEM((dk, f), jnp.float32),
            pltpu.VMEM((f, d), jnp.float32),
            pltpu.SemaphoreType.DMA((2,)),
            pltpu.SemaphoreType.DMA((3,)),
            pltpu.SemaphoreType.DMA((3,)),
        ],
        compiler_params=pltpu.CompilerParams(
            collective_id=0, vmem_limit_bytes=100 * 1024 * 1024),
    )(x, assign_col, W1, W2)


# device time: 111353 ns/iter; 1.3430x vs baseline; 1.3430x over previous
import jax
import jax.numpy as jnp
from jax import lax
from jax.experimental import pallas as pl
from jax.experimental.pallas import tpu as pltpu

N_LOCAL_EXPERTS = 2


def kernel(x, assign, W1, W2):
    t, d = x.shape
    _, dk, f = W1.shape
    assign_col = assign.reshape(t, 1)
    x_bf = x.astype(jnp.bfloat16)
    w1_bf = W1.astype(jnp.bfloat16)
    w2_bf = W2.astype(jnp.bfloat16)

    def body(x_ref, a_ref, w1_ref, w2_ref, out_ref,
             xr_ref, ar_ref, psend_ref, precv_ref, send_sems, recv_sems):
        my_x = lax.axis_index("x")
        my_y = lax.axis_index("y")
        my_z = lax.axis_index("z")
        partner = (1 - my_x, my_y, my_z)

        barrier = pltpu.get_barrier_semaphore()
        pl.semaphore_signal(barrier, inc=1, device_id=partner,
                            device_id_type=pl.DeviceIdType.MESH)
        pl.semaphore_wait(barrier, 1)

        send_x = pltpu.make_async_remote_copy(
            src_ref=x_ref, dst_ref=xr_ref,
            send_sem=send_sems.at[0], recv_sem=recv_sems.at[0],
            device_id=partner, device_id_type=pl.DeviceIdType.MESH)
        send_x.start()
        send_a = pltpu.make_async_remote_copy(
            src_ref=a_ref, dst_ref=ar_ref,
            send_sem=send_sems.at[1], recv_sem=recv_sems.at[1],
            device_id=partner, device_id_type=pl.DeviceIdType.MESH)
        send_a.start()

        e0 = my_x * N_LOCAL_EXPERTS

        def expert_partial(xs, acol, j):
            m = (acol == e0 + j)
            xm = jnp.where(m, xs, jnp.bfloat16(0))
            h = jnp.maximum(
                jnp.dot(xm, w1_ref[j], preferred_element_type=jnp.float32),
                0.0).astype(jnp.bfloat16)
            return jnp.dot(h, w2_ref[j], preferred_element_type=jnp.float32)

        out_ref[:, :] = expert_partial(x_ref[:, :], a_ref[:, :], 0)

        send_x.wait()
        send_a.wait()

        pr = (expert_partial(xr_ref[:, :], ar_ref[:, :], 0)
              + expert_partial(xr_ref[:, :], ar_ref[:, :], 1))
        psend_ref[:, :] = pr.astype(jnp.bfloat16)
        send_p = pltpu.make_async_remote_copy(
            src_ref=psend_ref, dst_ref=precv_ref,
            send_sem=send_sems.at[2], recv_sem=recv_sems.at[2],
            device_id=partner, device_id_type=pl.DeviceIdType.MESH)
        send_p.start()

        out_ref[:, :] = out_ref[:, :] + expert_partial(
            x_ref[:, :], a_ref[:, :], 1)

        send_p.wait()
        out_ref[:, :] = out_ref[:, :] + precv_ref[:, :].astype(jnp.float32)

    return pl.pallas_call(
        body,
        out_shape=jax.ShapeDtypeStruct((t, d), jnp.float32),
        in_specs=[pl.BlockSpec(memory_space=pltpu.VMEM)] * 4,
        out_specs=pl.BlockSpec(memory_space=pltpu.VMEM),
        scratch_shapes=[
            pltpu.VMEM((t, d), jnp.bfloat16),
            pltpu.VMEM((t, 1), jnp.int32),
            pltpu.VMEM((t, d), jnp.bfloat16),
            pltpu.VMEM((t, d), jnp.bfloat16),
            pltpu.SemaphoreType.DMA((3,)),
            pltpu.SemaphoreType.DMA((3,)),
        ],
        compiler_params=pltpu.CompilerParams(
            collective_id=0, vmem_limit_bytes=100 * 1024 * 1024),
    )(x_bf, assign_col, w1_bf, w2_bf)
